# baseline (speedup 1.0000x reference)
import functools

import jax
import jax.numpy as jnp
from jax import lax
from jax.experimental import pallas as pl
from jax.experimental.pallas import tpu as pltpu
from jax.experimental.pallas import tpu_sc as plsc

_NVOX = 128
_SLICE_IDX = 64
_SV = 1.0
_N = 20000

_DV = _SV / _NVOX
_X0 = -_SV / 2.0
_XS = _X0 + (_SLICE_IDX + 0.5) * _DV

_QCUT = 50.0
_NW = 32
_GPW = 640
_NPAD = _NW * _GPW
_RB = 8
_NB = _NVOX // _RB
_NF = 8
_CAP = _GPW + 32
_LISTW = _NB * _NF * _CAP


def _sc_bin_body(params_hbm, lists_hbm, counts_hbm, buf_v, lists_v, cnt_v,
                 pre_v):
    c = lax.axis_index("c")
    s = lax.axis_index("s")
    w = s * 2 + c
    pltpu.sync_copy(params_hbm.at[:, pl.ds(w * _GPW, _GPW)], buf_v)
    lane = lax.broadcasted_iota(jnp.int32, (16,), 0)
    pre_v[pl.ds(0, 16)] = jnp.zeros((16,), jnp.int32)
    zf0 = jnp.zeros((16,), jnp.float32)
    for b in range(_NB):
        for f in range(7):
            for o in range(0, 64, 16):
                lists_v[pl.ds((b * _NF + f) * _CAP + o, 16)] = zf0

    def body(i, curs):
        off = i * 16
        mx = buf_v[0, pl.ds(off, 16)]
        my = buf_v[1, pl.ds(off, 16)]
        mz = buf_v[2, pl.ds(off, 16)]
        op = buf_v[3, pl.ds(off, 16)]
        sx = buf_v[4, pl.ds(off, 16)]
        sy = buf_v[5, pl.ds(off, 16)]
        sz = buf_v[6, pl.ds(off, 16)]
        qw = buf_v[7, pl.ds(off, 16)]
        qx = buf_v[8, pl.ds(off, 16)]
        qy = buf_v[9, pl.ds(off, 16)]
        qz = buf_v[10, pl.ds(off, 16)]

        n2 = qw * qw + qx * qx + qy * qy + qz * qz
        r = 1.0 / n2
        xx = qx * qx
        yy = qy * qy
        zz = qz * qz
        xy = qx * qy
        xz = qx * qz
        yz = qy * qz
        wx = qw * qx
        wy = qw * qy
        wz = qw * qz
        r00 = 1.0 - 2.0 * r * (yy + zz)
        r01 = 2.0 * r * (xy - wz)
        r02 = 2.0 * r * (xz + wy)
        r10 = 2.0 * r * (xy + wz)
        r11 = 1.0 - 2.0 * r * (xx + zz)
        r12 = 2.0 * r * (yz - wx)
        r20 = 2.0 * r * (xz - wy)
        r21 = 2.0 * r * (yz + wx)
        r22 = 1.0 - 2.0 * r * (xx + yy)

        p1 = sx * sx
        p2 = sy * sy
        p3 = sz * sz
        i1 = 1.0 / p1
        i2 = 1.0 / p2
        i3 = 1.0 / p3

        a00 = i1 * r00 * r00 + i2 * r01 * r01 + i3 * r02 * r02
        a01 = i1 * r00 * r10 + i2 * r01 * r11 + i3 * r02 * r12
        a02 = i1 * r00 * r20 + i2 * r01 * r21 + i3 * r02 * r22
        a11 = i1 * r10 * r10 + i2 * r11 * r11 + i3 * r12 * r12
        a12 = i1 * r10 * r20 + i2 * r11 * r21 + i3 * r12 * r22
        a22 = i1 * r20 * r20 + i2 * r21 * r21 + i3 * r22 * r22

        sxx = p1 * r00 * r00 + p2 * r01 * r01 + p3 * r02 * r02
        sxy = p1 * r00 * r10 + p2 * r01 * r11 + p3 * r02 * r12
        syy = p1 * r10 * r10 + p2 * r11 * r11 + p3 * r12 * r12
        det_t = sxx * syy - sxy * sxy

        dx = _XS - mx
        dx2 = dx * dx
        margin = _QCUT * sxx - dx2
        keep = dx2 <= _QCUT * sxx
        cyv = (my + dx * sxy / sxx + 0.5) * _NVOX - 0.5
        hyv2 = det_t * margin / (sxx * sxx) * float(_NVOX * _NVOX)

        ayy = -0.5 * a11
        ayz = -0.5 * a12
        azz = -0.5 * a22
        by = -1.0 * a01 * dx
        bz = -1.0 * a02 * dx
        det2 = ayy * azz - ayz * ayz
        v0y = -0.5 * (azz * by - ayz * bz) / det2
        v0z = -0.5 * (ayy * bz - ayz * by) / det2
        ew = op * jnp.exp(-0.5 * dx2 / sxx)
        muy = my + v0y
        muz = mz + v0z
        cy = -2.0 * (ayy * muy + ayz * muz)
        cz = -2.0 * (ayz * muy + azz * muz)
        c1 = ayy * muy * muy + 2.0 * ayz * muy * muz + azz * muz * muz

        fields = (ew, ayy, 2.0 * ayz, azz, cy, cz, c1)

        new_curs = []
        for b in range(_NB):
            lo = float(b * _RB)
            hi = float(b * _RB + _RB - 1)
            d_lo = lo - cyv
            d_hi = cyv - hi
            c1 = (d_lo <= 0.0) | (d_lo * d_lo <= hyv2)
            c2 = (d_hi <= 0.0) | (d_hi * d_hi <= hyv2)
            mask = c1 & c2 & keep
            m32 = jnp.where(mask, jnp.ones((16,), jnp.int32),
                            jnp.zeros((16,), jnp.int32))
            v = m32
            for k in (1, 2, 4, 8):
                pre_v[pl.ds(16, 16)] = v
                v = v + pre_v[pl.ds(16 - k, 16)]
            cnt = v[15]
            excl = v - m32
            cur = curs[b]
            cur_vec = jnp.full((16,), cur, jnp.int32)
            slot0 = cur_vec + excl + (b * _NF * _CAP)
            for f in range(len(fields)):
                idx = jnp.where(mask, slot0 + f * _CAP, _LISTW + lane)
                plsc.store_scatter(lists_v, [idx], fields[f])
            new_curs.append(cur + cnt)
        return tuple(new_curs)

    curs = lax.fori_loop(0, _GPW // 16, body, (jnp.int32(0),) * _NB)

    zf = jnp.zeros((16,), jnp.float32)
    for b in range(_NB):
        for f in range(7):
            lists_v[pl.ds((b * _NF + f) * _CAP + curs[b], 16)] = zf
            lists_v[pl.ds((b * _NF + f) * _CAP + curs[b] + 16, 16)] = zf

    cv = jnp.zeros((16,), jnp.int32)
    for b in range(_NB):
        cv = jnp.where(lane == b, jnp.full((16,), curs[b], jnp.int32), cv)
    cnt_v[...] = cv

    pltpu.sync_copy(lists_v.at[pl.ds(0, _LISTW)], lists_hbm.at[w])
    pltpu.sync_copy(cnt_v, counts_hbm.at[w])


@functools.cache
def _get_sc_bin():
    return pl.kernel(
        _sc_bin_body,
        out_type=[jax.ShapeDtypeStruct((_NW, _LISTW), jnp.float32),
                  jax.ShapeDtypeStruct((_NW, 16), jnp.int32)],
        scratch_types=[pltpu.VMEM((11, _GPW), jnp.float32),
                       pltpu.VMEM((_LISTW + 16,), jnp.float32),
                       pltpu.VMEM((16,), jnp.int32),
                       pltpu.VMEM((32,), jnp.int32)],
        mesh=plsc.VectorSubcoreMesh(core_axis_name="c", subcore_axis_name="s"),
        compiler_params=pltpu.CompilerParams(needs_layout_passes=False),
    )


_PIX = _RB * _NVOX


def _tc_eval_kernel(lists_ref, counts_ref, out_ref):
    b = pl.program_id(0)
    li = lax.broadcasted_iota(jnp.int32, (1, _PIX), 1)
    yv = b * _RB + lax.shift_right_logical(li, 7)
    zv = li & 127
    y = _X0 + (yv.astype(jnp.float32) + 0.5) * _DV
    z = _X0 + (zv.astype(jnp.float32) + 0.5) * _DV
    zero = jnp.zeros_like(y)
    m8 = jnp.concatenate(
        [zero, y * y, y * z, z * z, y, z, jnp.ones_like(y), zero], axis=0)

    def chunk(grp, acc):
        qf = lax.dot_general(grp, m8, (((1,), (0,)), ((), ())),
                             precision=lax.Precision.HIGHEST,
                             preferred_element_type=jnp.float32)
        dens = grp[:, 0:1] * jnp.exp(qf)
        return acc + dens.reshape(-1, 8, _PIX).sum(axis=0)

    accs = [jnp.zeros((8, _PIX), jnp.float32) for _ in range(4)]
    for w in range(_NW):
        accs[w % 4] = chunk(lists_ref[0, 0:64, w * _NF:(w + 1) * _NF],
                            accs[w % 4])
    acc = (accs[0] + accs[1]) + (accs[2] + accs[3])
    for w in range(_NW):
        cnt = counts_ref[w, b]
        ng = (cnt + 31) // 32

        def body(j, acc, w=w):
            return chunk(lists_ref[0, pl.ds(j * 32, 32),
                                   w * _NF:(w + 1) * _NF], acc)

        acc = lax.fori_loop(2, jnp.maximum(ng, 2), body, acc)
    out_ref[0] = acc.sum(axis=0, keepdims=True)


def _radii_kernel(sc_ref, rad_ref):
    smax = jnp.max(sc_ref[...], axis=-1, keepdims=True)
    rad_ref[...] = jnp.ceil(3.0 * smax / _DV).astype(jnp.int32)


@jax.jit
def kernel(means3D, opacities, scales, rotations):
    params = jnp.concatenate(
        [means3D.T, opacities.T, scales.T, rotations.T], axis=0)
    pad_col = jnp.array([1e3, 0.0, 0.0, 0.0, 0.01, 0.01, 0.01,
                         1.0, 0.0, 0.0, 0.0], jnp.float32)[:, None]
    params = jnp.concatenate(
        [params, jnp.broadcast_to(pad_col, (11, _NPAD - _N))], axis=1)

    lists, counts = _get_sc_bin()(params)
    lists_t = (lists.reshape(_NW, _NB, _NF, _CAP)
               .transpose(1, 3, 0, 2)
               .reshape(_NB, _CAP, _NW * _NF))

    field = pl.pallas_call(
        _tc_eval_kernel,
        grid=(_NB,),
        in_specs=[
            pl.BlockSpec((1, _CAP, _NW * _NF), lambda b: (b, 0, 0)),
            pl.BlockSpec(memory_space=pltpu.SMEM),
        ],
        out_specs=pl.BlockSpec((1, 1, _PIX), lambda b: (b, 0, 0)),
        out_shape=jax.ShapeDtypeStruct((_NB, 1, _PIX), jnp.float32),
    )(lists_t, counts)
    field = field.reshape(_NVOX, _NVOX)

    radii = pl.pallas_call(
        _radii_kernel,
        out_shape=jax.ShapeDtypeStruct((_N, 1), jnp.int32),
    )(scales)

    return field[None, :, :], radii[:, 0]


# speedup vs baseline: 1.1343x; 1.1343x over previous
import functools

import jax
import jax.numpy as jnp
from jax import lax
from jax.experimental import pallas as pl
from jax.experimental.pallas import tpu as pltpu
from jax.experimental.pallas import tpu_sc as plsc

_NVOX = 128
_SLICE_IDX = 64
_SV = 1.0
_N = 20000

_DV = _SV / _NVOX
_X0 = -_SV / 2.0
_XS = _X0 + (_SLICE_IDX + 0.5) * _DV

_QCUT = 50.0
_NW = 32
_GPW = 640
_NPAD = _NW * _GPW
_RB = 8
_NB = _NVOX // _RB
_NF = 8
_CAP = _GPW + 32
_LISTW = _NB * _NF * _CAP


def _sc_bin_body(params_hbm, lists_hbm, counts_hbm, buf_v, lists_v, cnt_v,
                 pre_v):
    c = lax.axis_index("c")
    s = lax.axis_index("s")
    w = s * 2 + c
    pltpu.sync_copy(params_hbm.at[:, pl.ds(w * _GPW, _GPW)], buf_v)
    lane = lax.broadcasted_iota(jnp.int32, (16,), 0)
    pre_v[pl.ds(0, 16)] = jnp.zeros((16,), jnp.int32)
    zf0 = jnp.zeros((16,), jnp.float32)
    for b in range(_NB):
        for f in range(7):
            for o in range(0, 64, 16):
                lists_v[pl.ds((b * _NF + f) * _CAP + o, 16)] = zf0

    def body(i, curs):
        off = i * 16
        mx = buf_v[0, pl.ds(off, 16)]
        my = buf_v[1, pl.ds(off, 16)]
        mz = buf_v[2, pl.ds(off, 16)]
        op = buf_v[3, pl.ds(off, 16)]
        sx = buf_v[4, pl.ds(off, 16)]
        sy = buf_v[5, pl.ds(off, 16)]
        sz = buf_v[6, pl.ds(off, 16)]
        qw = buf_v[7, pl.ds(off, 16)]
        qx = buf_v[8, pl.ds(off, 16)]
        qy = buf_v[9, pl.ds(off, 16)]
        qz = buf_v[10, pl.ds(off, 16)]

        n2 = qw * qw + qx * qx + qy * qy + qz * qz
        r = 1.0 / n2
        xx = qx * qx
        yy = qy * qy
        zz = qz * qz
        xy = qx * qy
        xz = qx * qz
        yz = qy * qz
        wx = qw * qx
        wy = qw * qy
        wz = qw * qz
        r00 = 1.0 - 2.0 * r * (yy + zz)
        r01 = 2.0 * r * (xy - wz)
        r02 = 2.0 * r * (xz + wy)
        r10 = 2.0 * r * (xy + wz)
        r11 = 1.0 - 2.0 * r * (xx + zz)
        r12 = 2.0 * r * (yz - wx)
        r20 = 2.0 * r * (xz - wy)
        r21 = 2.0 * r * (yz + wx)
        r22 = 1.0 - 2.0 * r * (xx + yy)

        p1 = sx * sx
        p2 = sy * sy
        p3 = sz * sz
        i1 = 1.0 / p1
        i2 = 1.0 / p2
        i3 = 1.0 / p3

        a00 = i1 * r00 * r00 + i2 * r01 * r01 + i3 * r02 * r02
        a01 = i1 * r00 * r10 + i2 * r01 * r11 + i3 * r02 * r12
        a02 = i1 * r00 * r20 + i2 * r01 * r21 + i3 * r02 * r22
        a11 = i1 * r10 * r10 + i2 * r11 * r11 + i3 * r12 * r12
        a12 = i1 * r10 * r20 + i2 * r11 * r21 + i3 * r12 * r22
        a22 = i1 * r20 * r20 + i2 * r21 * r21 + i3 * r22 * r22

        sxx = p1 * r00 * r00 + p2 * r01 * r01 + p3 * r02 * r02
        sxy = p1 * r00 * r10 + p2 * r01 * r11 + p3 * r02 * r12
        syy = p1 * r10 * r10 + p2 * r11 * r11 + p3 * r12 * r12
        det_t = sxx * syy - sxy * sxy

        dx = _XS - mx
        dx2 = dx * dx
        margin = _QCUT * sxx - dx2
        keep = dx2 <= _QCUT * sxx
        cyv = (my + dx * sxy / sxx + 0.5) * _NVOX - 0.5
        hyv2 = det_t * margin / (sxx * sxx) * float(_NVOX * _NVOX)

        ayy = -0.5 * a11
        ayz = -0.5 * a12
        azz = -0.5 * a22
        by = -1.0 * a01 * dx
        bz = -1.0 * a02 * dx
        det2 = ayy * azz - ayz * ayz
        v0y = -0.5 * (azz * by - ayz * bz) / det2
        v0z = -0.5 * (ayy * bz - ayz * by) / det2
        ew = op * jnp.exp(-0.5 * dx2 / sxx)
        muy = my + v0y
        muz = mz + v0z
        cy = -2.0 * (ayy * muy + ayz * muz)
        cz = -2.0 * (ayz * muy + azz * muz)
        c1 = ayy * muy * muy + 2.0 * ayz * muy * muz + azz * muz * muz

        fields = (ew, ayy, 2.0 * ayz, azz, cy, cz, c1)

        new_curs = []
        for b in range(_NB):
            lo = float(b * _RB)
            hi = float(b * _RB + _RB - 1)
            d_lo = lo - cyv
            d_hi = cyv - hi
            c1 = (d_lo <= 0.0) | (d_lo * d_lo <= hyv2)
            c2 = (d_hi <= 0.0) | (d_hi * d_hi <= hyv2)
            mask = c1 & c2 & keep
            m32 = jnp.where(mask, jnp.ones((16,), jnp.int32),
                            jnp.zeros((16,), jnp.int32))
            v = m32
            for k in (1, 2, 4, 8):
                pre_v[pl.ds(16, 16)] = v
                v = v + pre_v[pl.ds(16 - k, 16)]
            cnt = v[15]
            excl = v - m32
            cur = curs[b]
            cur_vec = jnp.full((16,), cur, jnp.int32)
            slot0 = cur_vec + excl + (b * _NF * _CAP)
            for f in range(len(fields)):
                idx = jnp.where(mask, slot0 + f * _CAP, _LISTW + lane)
                plsc.store_scatter(lists_v, [idx], fields[f])
            new_curs.append(cur + cnt)
        return tuple(new_curs)

    curs = lax.fori_loop(0, _GPW // 16, body, (jnp.int32(0),) * _NB)

    zf = jnp.zeros((16,), jnp.float32)
    for b in range(_NB):
        for f in range(7):
            lists_v[pl.ds((b * _NF + f) * _CAP + curs[b], 16)] = zf
            lists_v[pl.ds((b * _NF + f) * _CAP + curs[b] + 16, 16)] = zf

    cv = jnp.zeros((16,), jnp.int32)
    for b in range(_NB):
        cv = jnp.where(lane == b, jnp.full((16,), curs[b], jnp.int32), cv)
    cnt_v[...] = cv

    pltpu.sync_copy(lists_v.at[pl.ds(0, _LISTW)], lists_hbm.at[w])
    pltpu.sync_copy(cnt_v, counts_hbm.at[w])


@functools.cache
def _get_sc_bin():
    return pl.kernel(
        _sc_bin_body,
        out_type=[jax.ShapeDtypeStruct((_NW, _LISTW), jnp.float32),
                  jax.ShapeDtypeStruct((_NW, 16), jnp.int32)],
        scratch_types=[pltpu.VMEM((11, _GPW), jnp.float32),
                       pltpu.VMEM((_LISTW + 16,), jnp.float32),
                       pltpu.VMEM((16,), jnp.int32),
                       pltpu.VMEM((32,), jnp.int32)],
        mesh=plsc.VectorSubcoreMesh(core_axis_name="c", subcore_axis_name="s"),
        compiler_params=pltpu.CompilerParams(needs_layout_passes=False),
    )


_PIX = _RB * _NVOX


def _tc_eval_kernel(lists_ref, counts_ref, out_ref):
    b = pl.program_id(0)
    li = lax.broadcasted_iota(jnp.int32, (1, _PIX), 1)
    yv = b * _RB + lax.shift_right_logical(li, 7)
    zv = li & 127
    y = _X0 + (yv.astype(jnp.float32) + 0.5) * _DV
    z = _X0 + (zv.astype(jnp.float32) + 0.5) * _DV
    zero = jnp.zeros_like(y)
    m8 = jnp.concatenate(
        [zero, y * y, y * z, z * z, y, z, jnp.ones_like(y), zero], axis=0)

    def chunk(grp, acc):
        qf = lax.dot_general(grp, m8, (((1,), (0,)), ((), ())),
                             precision=lax.Precision.HIGHEST,
                             preferred_element_type=jnp.float32)
        dens = grp[:, 0:1] * jnp.exp(qf)
        return acc + dens.reshape(-1, 8, _PIX).sum(axis=0)

    accs = [jnp.zeros((8, _PIX), jnp.float32) for _ in range(4)]
    for w in range(_NW):
        accs[w % 4] = chunk(lists_ref[0, 0:48, w * _NF:(w + 1) * _NF],
                            accs[w % 4])
    acc = (accs[0] + accs[1]) + (accs[2] + accs[3])
    for w in range(_NW):
        cnt = counts_ref[w, b]
        ng = jnp.maximum((cnt - 48 + 31) // 32, 0)

        def body(j, acc, w=w):
            return chunk(lists_ref[0, pl.ds(48 + j * 32, 32),
                                   w * _NF:(w + 1) * _NF], acc)

        acc = lax.fori_loop(0, ng, body, acc)
    out_ref[0] = acc.sum(axis=0, keepdims=True)


def _radii_kernel(sc_ref, rad_ref):
    smax = jnp.max(sc_ref[...], axis=-1, keepdims=True)
    rad_ref[...] = jnp.ceil(3.0 * smax / _DV).astype(jnp.int32)


@jax.jit
def kernel(means3D, opacities, scales, rotations):
    params = jnp.concatenate(
        [means3D.T, opacities.T, scales.T, rotations.T], axis=0)
    pad_col = jnp.array([1e3, 0.0, 0.0, 0.0, 0.01, 0.01, 0.01,
                         1.0, 0.0, 0.0, 0.0], jnp.float32)[:, None]
    params = jnp.concatenate(
        [params, jnp.broadcast_to(pad_col, (11, _NPAD - _N))], axis=1)

    lists, counts = _get_sc_bin()(params)
    lists_t = (lists.reshape(_NW, _NB, _NF, _CAP)
               .transpose(1, 3, 0, 2)
               .reshape(_NB, _CAP, _NW * _NF))

    field = pl.pallas_call(
        _tc_eval_kernel,
        grid=(_NB,),
        in_specs=[
            pl.BlockSpec((1, _CAP, _NW * _NF), lambda b: (b, 0, 0)),
            pl.BlockSpec(memory_space=pltpu.SMEM),
        ],
        out_specs=pl.BlockSpec((1, 1, _PIX), lambda b: (b, 0, 0)),
        out_shape=jax.ShapeDtypeStruct((_NB, 1, _PIX), jnp.float32),
    )(lists_t, counts)
    field = field.reshape(_NVOX, _NVOX)

    radii = pl.pallas_call(
        _radii_kernel,
        out_shape=jax.ShapeDtypeStruct((_N, 1), jnp.int32),
    )(scales)

    return field[None, :, :], radii[:, 0]


# speedup vs baseline: 1.2128x; 1.0692x over previous
import functools

import jax
import jax.numpy as jnp
from jax import lax
from jax.experimental import pallas as pl
from jax.experimental.pallas import tpu as pltpu
from jax.experimental.pallas import tpu_sc as plsc

_NVOX = 128
_SLICE_IDX = 64
_SV = 1.0
_N = 20000

_DV = _SV / _NVOX
_X0 = -_SV / 2.0
_XS = _X0 + (_SLICE_IDX + 0.5) * _DV

_QCUT = 36.0
_NW = 32
_GPW = 640
_NPAD = _NW * _GPW
_RB = 8
_NB = _NVOX // _RB
_NF = 8
_CAP = _GPW + 32
_LISTW = _NB * _NF * _CAP


def _sc_bin_body(params_hbm, lists_hbm, counts_hbm, buf_v, lists_v, cnt_v,
                 pre_v):
    c = lax.axis_index("c")
    s = lax.axis_index("s")
    w = s * 2 + c
    pltpu.sync_copy(params_hbm.at[:, pl.ds(w * _GPW, _GPW)], buf_v)
    lane = lax.broadcasted_iota(jnp.int32, (16,), 0)
    pre_v[pl.ds(0, 16)] = jnp.zeros((16,), jnp.int32)
    zf0 = jnp.zeros((16,), jnp.float32)
    for b in range(_NB):
        for f in range(7):
            for o in range(0, 64, 16):
                lists_v[pl.ds((b * _NF + f) * _CAP + o, 16)] = zf0

    def body(i, curs):
        off = i * 16
        mx = buf_v[0, pl.ds(off, 16)]
        my = buf_v[1, pl.ds(off, 16)]
        mz = buf_v[2, pl.ds(off, 16)]
        op = buf_v[3, pl.ds(off, 16)]
        sx = buf_v[4, pl.ds(off, 16)]
        sy = buf_v[5, pl.ds(off, 16)]
        sz = buf_v[6, pl.ds(off, 16)]
        qw = buf_v[7, pl.ds(off, 16)]
        qx = buf_v[8, pl.ds(off, 16)]
        qy = buf_v[9, pl.ds(off, 16)]
        qz = buf_v[10, pl.ds(off, 16)]

        n2 = qw * qw + qx * qx + qy * qy + qz * qz
        r = 1.0 / n2
        xx = qx * qx
        yy = qy * qy
        zz = qz * qz
        xy = qx * qy
        xz = qx * qz
        yz = qy * qz
        wx = qw * qx
        wy = qw * qy
        wz = qw * qz
        r00 = 1.0 - 2.0 * r * (yy + zz)
        r01 = 2.0 * r * (xy - wz)
        r02 = 2.0 * r * (xz + wy)
        r10 = 2.0 * r * (xy + wz)
        r11 = 1.0 - 2.0 * r * (xx + zz)
        r12 = 2.0 * r * (yz - wx)
        r20 = 2.0 * r * (xz - wy)
        r21 = 2.0 * r * (yz + wx)
        r22 = 1.0 - 2.0 * r * (xx + yy)

        p1 = sx * sx
        p2 = sy * sy
        p3 = sz * sz
        i1 = 1.0 / p1
        i2 = 1.0 / p2
        i3 = 1.0 / p3

        a00 = i1 * r00 * r00 + i2 * r01 * r01 + i3 * r02 * r02
        a01 = i1 * r00 * r10 + i2 * r01 * r11 + i3 * r02 * r12
        a02 = i1 * r00 * r20 + i2 * r01 * r21 + i3 * r02 * r22
        a11 = i1 * r10 * r10 + i2 * r11 * r11 + i3 * r12 * r12
        a12 = i1 * r10 * r20 + i2 * r11 * r21 + i3 * r12 * r22
        a22 = i1 * r20 * r20 + i2 * r21 * r21 + i3 * r22 * r22

        sxx = p1 * r00 * r00 + p2 * r01 * r01 + p3 * r02 * r02
        sxy = p1 * r00 * r10 + p2 * r01 * r11 + p3 * r02 * r12
        syy = p1 * r10 * r10 + p2 * r11 * r11 + p3 * r12 * r12
        det_t = sxx * syy - sxy * sxy

        dx = _XS - mx
        dx2 = dx * dx
        margin = _QCUT * sxx - dx2
        keep = dx2 <= _QCUT * sxx
        cyv = (my + dx * sxy / sxx + 0.5) * _NVOX - 0.5
        hyv2 = det_t * margin / (sxx * sxx) * float(_NVOX * _NVOX)

        ayy = -0.5 * a11
        ayz = -0.5 * a12
        azz = -0.5 * a22
        by = -1.0 * a01 * dx
        bz = -1.0 * a02 * dx
        det2 = ayy * azz - ayz * ayz
        v0y = -0.5 * (azz * by - ayz * bz) / det2
        v0z = -0.5 * (ayy * bz - ayz * by) / det2
        ew = op * jnp.exp(-0.5 * dx2 / sxx)
        muy = my + v0y
        muz = mz + v0z
        cy = -2.0 * (ayy * muy + ayz * muz)
        cz = -2.0 * (ayz * muy + azz * muz)
        c1 = ayy * muy * muy + 2.0 * ayz * muy * muz + azz * muz * muz

        fields = (ew, ayy, 2.0 * ayz, azz, cy, cz, c1)

        new_curs = []
        for b in range(_NB):
            lo = float(b * _RB)
            hi = float(b * _RB + _RB - 1)
            d_lo = lo - cyv
            d_hi = cyv - hi
            c1 = (d_lo <= 0.0) | (d_lo * d_lo <= hyv2)
            c2 = (d_hi <= 0.0) | (d_hi * d_hi <= hyv2)
            mask = c1 & c2 & keep
            m32 = jnp.where(mask, jnp.ones((16,), jnp.int32),
                            jnp.zeros((16,), jnp.int32))
            v = m32
            for k in (1, 2, 4, 8):
                pre_v[pl.ds(16, 16)] = v
                v = v + pre_v[pl.ds(16 - k, 16)]
            cnt = v[15]
            excl = v - m32
            cur = curs[b]
            cur_vec = jnp.full((16,), cur, jnp.int32)
            slot0 = cur_vec + excl + (b * _NF * _CAP)
            for f in range(len(fields)):
                idx = jnp.where(mask, slot0 + f * _CAP, _LISTW + lane)
                plsc.store_scatter(lists_v, [idx], fields[f])
            new_curs.append(cur + cnt)
        return tuple(new_curs)

    curs = lax.fori_loop(0, _GPW // 16, body, (jnp.int32(0),) * _NB)

    zf = jnp.zeros((16,), jnp.float32)
    for b in range(_NB):
        for f in range(7):
            lists_v[pl.ds((b * _NF + f) * _CAP + curs[b], 16)] = zf
            lists_v[pl.ds((b * _NF + f) * _CAP + curs[b] + 16, 16)] = zf

    cv = jnp.zeros((16,), jnp.int32)
    for b in range(_NB):
        cv = jnp.where(lane == b, jnp.full((16,), curs[b], jnp.int32), cv)
    cnt_v[...] = cv

    pltpu.sync_copy(lists_v.at[pl.ds(0, _LISTW)], lists_hbm.at[w])
    pltpu.sync_copy(cnt_v, counts_hbm.at[w])


@functools.cache
def _get_sc_bin():
    return pl.kernel(
        _sc_bin_body,
        out_type=[jax.ShapeDtypeStruct((_NW, _LISTW), jnp.float32),
                  jax.ShapeDtypeStruct((_NW, 16), jnp.int32)],
        scratch_types=[pltpu.VMEM((11, _GPW), jnp.float32),
                       pltpu.VMEM((_LISTW + 16,), jnp.float32),
                       pltpu.VMEM((16,), jnp.int32),
                       pltpu.VMEM((32,), jnp.int32)],
        mesh=plsc.VectorSubcoreMesh(core_axis_name="c", subcore_axis_name="s"),
        compiler_params=pltpu.CompilerParams(needs_layout_passes=False),
    )


_PIX = _RB * _NVOX


def _tc_eval_kernel(lists_ref, counts_ref, out_ref):
    b = pl.program_id(0)
    li = lax.broadcasted_iota(jnp.int32, (1, _PIX), 1)
    yv = b * _RB + lax.shift_right_logical(li, 7)
    zv = li & 127
    y = _X0 + (yv.astype(jnp.float32) + 0.5) * _DV
    z = _X0 + (zv.astype(jnp.float32) + 0.5) * _DV
    zero = jnp.zeros_like(y)
    m8 = jnp.concatenate(
        [zero, y * y, y * z, z * z, y, z, jnp.ones_like(y), zero], axis=0)

    def chunk(grp, acc):
        qf = lax.dot_general(grp, m8, (((1,), (0,)), ((), ())),
                             precision=lax.Precision.HIGHEST,
                             preferred_element_type=jnp.float32)
        dens = grp[:, 0:1] * jnp.exp(qf)
        return acc + dens.reshape(-1, 8, _PIX).sum(axis=0)

    accs = [jnp.zeros((8, _PIX), jnp.float32) for _ in range(4)]
    for w in range(_NW):
        accs[w % 4] = chunk(lists_ref[0, 0:40, w * _NF:(w + 1) * _NF],
                            accs[w % 4])
    acc = (accs[0] + accs[1]) + (accs[2] + accs[3])
    for w in range(_NW):
        cnt = counts_ref[w, b]
        ng = jnp.maximum((cnt - 40 + 31) // 32, 0)

        def body(j, acc, w=w):
            return chunk(lists_ref[0, pl.ds(40 + j * 32, 32),
                                   w * _NF:(w + 1) * _NF], acc)

        acc = lax.fori_loop(0, ng, body, acc)
    out_ref[0] = acc.sum(axis=0, keepdims=True)


def _radii_kernel(sc_ref, rad_ref):
    smax = jnp.max(sc_ref[...], axis=-1, keepdims=True)
    rad_ref[...] = jnp.ceil(3.0 * smax / _DV).astype(jnp.int32)


@jax.jit
def kernel(means3D, opacities, scales, rotations):
    params = jnp.concatenate(
        [means3D.T, opacities.T, scales.T, rotations.T], axis=0)
    pad_col = jnp.array([1e3, 0.0, 0.0, 0.0, 0.01, 0.01, 0.01,
                         1.0, 0.0, 0.0, 0.0], jnp.float32)[:, None]
    params = jnp.concatenate(
        [params, jnp.broadcast_to(pad_col, (11, _NPAD - _N))], axis=1)

    lists, counts = _get_sc_bin()(params)
    lists_t = (lists.reshape(_NW, _NB, _NF, _CAP)
               .transpose(1, 3, 0, 2)
               .reshape(_NB, _CAP, _NW * _NF))

    field = pl.pallas_call(
        _tc_eval_kernel,
        grid=(_NB,),
        in_specs=[
            pl.BlockSpec((1, _CAP, _NW * _NF), lambda b: (b, 0, 0)),
            pl.BlockSpec(memory_space=pltpu.SMEM),
        ],
        out_specs=pl.BlockSpec((1, 1, _PIX), lambda b: (b, 0, 0)),
        out_shape=jax.ShapeDtypeStruct((_NB, 1, _PIX), jnp.float32),
    )(lists_t, counts)
    field = field.reshape(_NVOX, _NVOX)

    radii = pl.pallas_call(
        _radii_kernel,
        out_shape=jax.ShapeDtypeStruct((_N, 1), jnp.int32),
    )(scales)

    return field[None, :, :], radii[:, 0]


# speedup vs baseline: 1.2863x; 1.0606x over previous
import functools

import jax
import jax.numpy as jnp
from jax import lax
from jax.experimental import pallas as pl
from jax.experimental.pallas import tpu as pltpu
from jax.experimental.pallas import tpu_sc as plsc

_NVOX = 128
_SLICE_IDX = 64
_SV = 1.0
_N = 20000

_DV = _SV / _NVOX
_X0 = -_SV / 2.0
_XS = _X0 + (_SLICE_IDX + 0.5) * _DV

_QCUT = 36.0
_NW = 32
_GPW = 640
_NPAD = _NW * _GPW
_RB = 8
_NB = _NVOX // _RB
_NF = 8
_CAP = _GPW + 32
_LISTW = _NB * _NF * _CAP


def _sc_bin_body(params_hbm, lists_hbm, counts_hbm, buf_v, lists_v, cnt_v,
                 pre_v):
    c = lax.axis_index("c")
    s = lax.axis_index("s")
    w = s * 2 + c
    pltpu.sync_copy(params_hbm.at[:, pl.ds(w * _GPW, _GPW)], buf_v)
    lane = lax.broadcasted_iota(jnp.int32, (16,), 0)
    pre_v[pl.ds(0, 16)] = jnp.zeros((16,), jnp.int32)
    zf0 = jnp.zeros((16,), jnp.float32)
    for b in range(_NB):
        for f in range(7):
            for o in range(0, 64, 16):
                lists_v[pl.ds((b * _NF + f) * _CAP + o, 16)] = zf0

    def body(i, curs):
        off = i * 16
        mx = buf_v[0, pl.ds(off, 16)]
        my = buf_v[1, pl.ds(off, 16)]
        mz = buf_v[2, pl.ds(off, 16)]
        op = buf_v[3, pl.ds(off, 16)]
        sx = buf_v[4, pl.ds(off, 16)]
        sy = buf_v[5, pl.ds(off, 16)]
        sz = buf_v[6, pl.ds(off, 16)]
        qw = buf_v[7, pl.ds(off, 16)]
        qx = buf_v[8, pl.ds(off, 16)]
        qy = buf_v[9, pl.ds(off, 16)]
        qz = buf_v[10, pl.ds(off, 16)]

        n2 = qw * qw + qx * qx + qy * qy + qz * qz
        r = 1.0 / n2
        xx = qx * qx
        yy = qy * qy
        zz = qz * qz
        xy = qx * qy
        xz = qx * qz
        yz = qy * qz
        wx = qw * qx
        wy = qw * qy
        wz = qw * qz
        r00 = 1.0 - 2.0 * r * (yy + zz)
        r01 = 2.0 * r * (xy - wz)
        r02 = 2.0 * r * (xz + wy)
        r10 = 2.0 * r * (xy + wz)
        r11 = 1.0 - 2.0 * r * (xx + zz)
        r12 = 2.0 * r * (yz - wx)
        r20 = 2.0 * r * (xz - wy)
        r21 = 2.0 * r * (yz + wx)
        r22 = 1.0 - 2.0 * r * (xx + yy)

        p1 = sx * sx
        p2 = sy * sy
        p3 = sz * sz
        i1 = 1.0 / p1
        i2 = 1.0 / p2
        i3 = 1.0 / p3

        a00 = i1 * r00 * r00 + i2 * r01 * r01 + i3 * r02 * r02
        a01 = i1 * r00 * r10 + i2 * r01 * r11 + i3 * r02 * r12
        a02 = i1 * r00 * r20 + i2 * r01 * r21 + i3 * r02 * r22
        a11 = i1 * r10 * r10 + i2 * r11 * r11 + i3 * r12 * r12
        a12 = i1 * r10 * r20 + i2 * r11 * r21 + i3 * r12 * r22
        a22 = i1 * r20 * r20 + i2 * r21 * r21 + i3 * r22 * r22

        sxx = p1 * r00 * r00 + p2 * r01 * r01 + p3 * r02 * r02
        sxy = p1 * r00 * r10 + p2 * r01 * r11 + p3 * r02 * r12
        syy = p1 * r10 * r10 + p2 * r11 * r11 + p3 * r12 * r12
        det_t = sxx * syy - sxy * sxy

        dx = _XS - mx
        dx2 = dx * dx
        margin = _QCUT * sxx - dx2
        keep = dx2 <= _QCUT * sxx
        cyv = (my + dx * sxy / sxx + 0.5) * _NVOX - 0.5
        hyv2 = det_t * margin / (sxx * sxx) * float(_NVOX * _NVOX)

        ayy = -0.5 * a11
        ayz = -0.5 * a12
        azz = -0.5 * a22
        by = -1.0 * a01 * dx
        bz = -1.0 * a02 * dx
        det2 = ayy * azz - ayz * ayz
        v0y = -0.5 * (azz * by - ayz * bz) / det2
        v0z = -0.5 * (ayy * bz - ayz * by) / det2
        ew = op * jnp.exp(-0.5 * dx2 / sxx)
        muy = my + v0y
        muz = mz + v0z
        cy = -2.0 * (ayy * muy + ayz * muz)
        cz = -2.0 * (ayz * muy + azz * muz)
        c1 = ayy * muy * muy + 2.0 * ayz * muy * muz + azz * muz * muz

        fields = (ew, ayy, 2.0 * ayz, azz, cy, cz, c1)

        new_curs = []
        for b in range(_NB):
            lo = float(b * _RB)
            hi = float(b * _RB + _RB - 1)
            d_lo = lo - cyv
            d_hi = cyv - hi
            c1 = (d_lo <= 0.0) | (d_lo * d_lo <= hyv2)
            c2 = (d_hi <= 0.0) | (d_hi * d_hi <= hyv2)
            mask = c1 & c2 & keep
            m32 = jnp.where(mask, jnp.ones((16,), jnp.int32),
                            jnp.zeros((16,), jnp.int32))
            v = m32
            for k in (1, 2, 4, 8):
                pre_v[pl.ds(16, 16)] = v
                v = v + pre_v[pl.ds(16 - k, 16)]
            cnt = v[15]
            excl = v - m32
            cur = curs[b]
            cur_vec = jnp.full((16,), cur, jnp.int32)
            slot0 = cur_vec + excl + (b * _NF * _CAP)
            for f in range(len(fields)):
                idx = jnp.where(mask, slot0 + f * _CAP, _LISTW + lane)
                plsc.store_scatter(lists_v, [idx], fields[f])
            new_curs.append(cur + cnt)
        return tuple(new_curs)

    curs = lax.fori_loop(0, _GPW // 16, body, (jnp.int32(0),) * _NB)

    zf = jnp.zeros((16,), jnp.float32)
    for b in range(_NB):
        for f in range(7):
            lists_v[pl.ds((b * _NF + f) * _CAP + curs[b], 16)] = zf
            lists_v[pl.ds((b * _NF + f) * _CAP + curs[b] + 16, 16)] = zf

    cv = jnp.zeros((16,), jnp.int32)
    for b in range(_NB):
        cv = jnp.where(lane == b, jnp.full((16,), curs[b], jnp.int32), cv)
    cnt_v[...] = cv

    pltpu.sync_copy(lists_v.at[pl.ds(0, _LISTW)], lists_hbm.at[w])
    pltpu.sync_copy(cnt_v, counts_hbm.at[w])


@functools.cache
def _get_sc_bin():
    return pl.kernel(
        _sc_bin_body,
        out_type=[jax.ShapeDtypeStruct((_NW, _LISTW), jnp.float32),
                  jax.ShapeDtypeStruct((_NW, 16), jnp.int32)],
        scratch_types=[pltpu.VMEM((11, _GPW), jnp.float32),
                       pltpu.VMEM((_LISTW + 16,), jnp.float32),
                       pltpu.VMEM((16,), jnp.int32),
                       pltpu.VMEM((32,), jnp.int32)],
        mesh=plsc.VectorSubcoreMesh(core_axis_name="c", subcore_axis_name="s"),
        compiler_params=pltpu.CompilerParams(needs_layout_passes=False),
    )


_PIX = _RB * _NVOX


def _tc_eval_kernel(lists_ref, counts_ref, out_ref):
    b = pl.program_id(0)
    li = lax.broadcasted_iota(jnp.int32, (1, _PIX), 1)
    yv = b * _RB + lax.shift_right_logical(li, 7)
    zv = li & 127
    y = _X0 + (yv.astype(jnp.float32) + 0.5) * _DV
    z = _X0 + (zv.astype(jnp.float32) + 0.5) * _DV
    zero = jnp.zeros_like(y)
    m8 = jnp.concatenate(
        [zero, y * y, y * z, z * z, y, z, jnp.ones_like(y), zero], axis=0)

    def chunk(grp, acc):
        qf = lax.dot_general(grp, m8, (((1,), (0,)), ((), ())),
                             precision=lax.Precision.HIGHEST,
                             preferred_element_type=jnp.float32)
        dens = grp[:, 0:1] * jnp.exp(qf)
        return acc + dens.reshape(-1, 8, _PIX).sum(axis=0)

    accs = [jnp.zeros((8, _PIX), jnp.float32) for _ in range(4)]
    for w in range(_NW):
        accs[w % 4] = chunk(lists_ref[0, 0:32, w * _NF:(w + 1) * _NF],
                            accs[w % 4])
    acc = (accs[0] + accs[1]) + (accs[2] + accs[3])
    for w in range(_NW):
        cnt = counts_ref[w, b]
        ng = jnp.maximum((cnt - 32 + 31) // 32, 0)

        def body(j, acc, w=w):
            return chunk(lists_ref[0, pl.ds(32 + j * 32, 32),
                                   w * _NF:(w + 1) * _NF], acc)

        acc = lax.fori_loop(0, ng, body, acc)
    out_ref[0] = acc.sum(axis=0, keepdims=True)


def _radii_kernel(sc_ref, rad_ref):
    smax = jnp.max(sc_ref[...], axis=-1, keepdims=True)
    rad_ref[...] = jnp.ceil(3.0 * smax / _DV).astype(jnp.int32)


@jax.jit
def kernel(means3D, opacities, scales, rotations):
    params = jnp.concatenate(
        [means3D.T, opacities.T, scales.T, rotations.T], axis=0)
    pad_col = jnp.array([1e3, 0.0, 0.0, 0.0, 0.01, 0.01, 0.01,
                         1.0, 0.0, 0.0, 0.0], jnp.float32)[:, None]
    params = jnp.concatenate(
        [params, jnp.broadcast_to(pad_col, (11, _NPAD - _N))], axis=1)

    lists, counts = _get_sc_bin()(params)
    lists_t = (lists.reshape(_NW, _NB, _NF, _CAP)
               .transpose(1, 3, 0, 2)
               .reshape(_NB, _CAP, _NW * _NF))

    field = pl.pallas_call(
        _tc_eval_kernel,
        grid=(_NB,),
        in_specs=[
            pl.BlockSpec((1, _CAP, _NW * _NF), lambda b: (b, 0, 0)),
            pl.BlockSpec(memory_space=pltpu.SMEM),
        ],
        out_specs=pl.BlockSpec((1, 1, _PIX), lambda b: (b, 0, 0)),
        out_shape=jax.ShapeDtypeStruct((_NB, 1, _PIX), jnp.float32),
    )(lists_t, counts)
    field = field.reshape(_NVOX, _NVOX)

    radii = pl.pallas_call(
        _radii_kernel,
        out_shape=jax.ShapeDtypeStruct((_N, 1), jnp.int32),
    )(scales)

    return field[None, :, :], radii[:, 0]
